# Initial kernel scaffold; baseline (speedup 1.0000x reference)
import jax
import jax.numpy as jnp
from jax.experimental import pallas as pl


def kernel(pred, target):
    raise NotImplementedError("write your pallas kernel here")


# speedup vs baseline: 3.0654x; 3.0654x over previous
import jax
import jax.numpy as jnp
import numpy as np
from jax.experimental import pallas as pl
from jax.experimental.pallas import tpu as pltpu

_ANCHORS = np.array(
    [[0.57273, 0.677385], [1.87446, 2.06253], [3.33843, 5.47434],
     [7.88282, 3.52778], [9.77052, 9.16828]], dtype=np.float32)

_A = 5
_CH = 85
_NCLS = _CH - 5


def _softplus(x):
    return jnp.maximum(x, 0.0) + jnp.log1p(jnp.exp(-jnp.abs(x)))


def _loss_body(p_ref, t_ref, o_ref):
    aw = jnp.concatenate(
        [jnp.full((1, 1), float(v), jnp.float32) for v in _ANCHORS[:, 0]], axis=0)
    ah = jnp.concatenate(
        [jnp.full((1, 1), float(v), jnp.float32) for v in _ANCHORS[:, 1]], axis=0)

    p_tx = p_ref[0:5, :]
    p_ty = p_ref[5:10, :]
    p_tw = p_ref[10:15, :]
    p_th = p_ref[15:20, :]
    p_to = p_ref[20:25, :]
    t_tx = t_ref[0:5, :]
    t_ty = t_ref[5:10, :]
    t_tw = t_ref[10:15, :]
    t_th = t_ref[15:20, :]
    t_to = t_ref[20:25, :]

    psx = jax.nn.sigmoid(p_tx)
    psy = jax.nn.sigmoid(p_ty)
    pw = jnp.exp(p_tw) * aw
    ph = jnp.exp(p_th) * ah
    gw = jnp.exp(t_tw) * aw
    gh = jnp.exp(t_th) * ah

    px1 = psx - pw * 0.5
    px2 = psx + pw * 0.5
    py1 = psy - ph * 0.5
    py2 = psy + ph * 0.5
    parea = (px2 - px1) * (py2 - py1)

    gx1 = t_tx - gw * 0.5
    gx2 = t_tx + gw * 0.5
    gy1 = t_ty - gh * 0.5
    gy2 = t_ty + gh * 0.5
    garea = (gx2 - gx1) * (gy2 - gy1)

    takenf = jnp.zeros(p_tx.shape, dtype=jnp.float32)
    updfs = []
    for g in range(_A):
        ix1 = jnp.maximum(px1, gx1[g:g + 1, :])
        iy1 = jnp.maximum(py1, gy1[g:g + 1, :])
        ix2 = jnp.minimum(px2, gx2[g:g + 1, :])
        iy2 = jnp.minimum(py2, gy2[g:g + 1, :])
        iw = jnp.maximum(ix2 - ix1, 0.0)
        ih = jnp.maximum(iy2 - iy1, 0.0)
        inter = iw * ih
        union = parea + garea[g:g + 1, :] - inter + 1e-09
        iou = inter / union
        iou = jnp.where(takenf > 0.5, -1.0, iou)
        m = jnp.max(iou, axis=0, keepdims=True)
        e = (iou == m).astype(jnp.float32)
        seen = e[0:1, :]
        rows = [seen]
        for a in range(1, _A):
            rows.append(e[a:a + 1, :] * (1.0 - seen))
            seen = jnp.maximum(seen, e[a:a + 1, :])
        oh = jnp.concatenate(rows, axis=0)
        isobj = (t_to[g:g + 1, :] > 0.5).astype(jnp.float32)
        upd = oh * isobj
        takenf = takenf + upd
        updfs.append(upd)

    objf = takenf

    def gather_tgt(tch):
        s = updfs[0] * tch[0:1, :]
        for g in range(1, _A):
            s = s + updfs[g] * tch[g:g + 1, :]
        return s

    alx = gather_tgt(t_tx)
    aly = gather_tgt(t_ty)
    alw = gather_tgt(t_tw)
    alh = gather_tgt(t_th)

    sxy = jnp.sum(objf * ((psx - alx) ** 2 + (psy - aly) ** 2))
    swh = jnp.sum(objf * ((p_tw - alw) ** 2 + (p_th - alh) ** 2))
    sobj = jnp.sum(objf * _softplus(-p_to))
    snoobj = jnp.sum((1.0 - objf) * _softplus(p_to))

    amax = None
    psel = None
    pmax = None
    for k in range(_NCLS):
        r0 = 25 + 5 * k
        pk = p_ref[r0:r0 + 5, :]
        tk = t_ref[r0:r0 + 5, :]
        ak = gather_tgt(tk)
        if k == 0:
            amax, psel, pmax = ak, pk, pk
        else:
            better = ak > amax
            amax = jnp.where(better, ak, amax)
            psel = jnp.where(better, pk, psel)
            pmax = jnp.maximum(pmax, pk)
    se = jnp.zeros(pmax.shape, dtype=jnp.float32)
    for k in range(_NCLS):
        r0 = 25 + 5 * k
        se = se + jnp.exp(p_ref[r0:r0 + 5, :] - pmax)
    lse = pmax + jnp.log(se)
    scls = jnp.sum(objf * (lse - psel))

    @pl.when(pl.program_id(0) == 0)
    def _init():
        for i in range(5):
            o_ref[i] = 0.0

    o_ref[0] += sxy
    o_ref[1] += swh
    o_ref[2] += sobj
    o_ref[3] += snoobj
    o_ref[4] += scls


def _make_call(C, CB, interpret=False):
    grid = C // CB
    return pl.pallas_call(
        _loss_body,
        grid=(grid,),
        in_specs=[
            pl.BlockSpec((_CH * _A, CB), lambda i: (0, i)),
            pl.BlockSpec((_CH * _A, CB), lambda i: (0, i)),
        ],
        out_specs=pl.BlockSpec(memory_space=pltpu.SMEM),
        out_shape=jax.ShapeDtypeStruct((5,), jnp.float32),
        interpret=interpret,
    )


def kernel(pred, target):
    N, S, _, A, ch = pred.shape
    C = N * S * S
    pt = jnp.transpose(pred, (4, 3, 0, 1, 2)).reshape(ch * A, C)
    tt = jnp.transpose(target, (4, 3, 0, 1, 2)).reshape(ch * A, C)
    CB = 1664 if C % 1664 == 0 else C
    sums = _make_call(C, CB)(pt, tt)
    n = jnp.float32(N)
    lxy = 5.0 * sums[0] / n
    lwh = 5.0 * sums[1] / n
    lobj = 1.0 * sums[2] / n
    lnoobj = 0.5 * sums[3] / n
    lcls = 1.0 * sums[4] / n
    total = lxy + lwh + lobj + lnoobj + lcls
    return (total, lxy, lwh, lobj, lnoobj, lcls)


# speedup vs baseline: 4.1126x; 1.3416x over previous
import jax
import jax.numpy as jnp
import numpy as np
from jax.experimental import pallas as pl
from jax.experimental.pallas import tpu as pltpu

_ANCHORS = np.array(
    [[0.57273, 0.677385], [1.87446, 2.06253], [3.33843, 5.47434],
     [7.88282, 3.52778], [9.77052, 9.16828]], dtype=np.float32)

_A = 5
_CH = 85
_NCLS = _CH - 5


def _softplus(x):
    return jnp.maximum(x, 0.0) + jnp.log1p(jnp.exp(-jnp.abs(x)))


def _rows(mat, rows_idx):
    return jnp.concatenate([mat[r:r + 1, :] for r in rows_idx], axis=0)


def _loss_body(p_ref, t_ref, o_ref):
    P = jnp.transpose(p_ref[...], (1, 0))
    T = jnp.transpose(t_ref[...], (1, 0))

    aw = jnp.concatenate(
        [jnp.full((1, 1), float(v), jnp.float32) for v in _ANCHORS[:, 0]], axis=0)
    ah = jnp.concatenate(
        [jnp.full((1, 1), float(v), jnp.float32) for v in _ANCHORS[:, 1]], axis=0)

    p_tx = _rows(P, [85 * a + 0 for a in range(_A)])
    p_ty = _rows(P, [85 * a + 1 for a in range(_A)])
    p_tw = _rows(P, [85 * a + 2 for a in range(_A)])
    p_th = _rows(P, [85 * a + 3 for a in range(_A)])
    p_to = _rows(P, [85 * a + 4 for a in range(_A)])
    t_tx = _rows(T, [85 * a + 0 for a in range(_A)])
    t_ty = _rows(T, [85 * a + 1 for a in range(_A)])
    t_tw = _rows(T, [85 * a + 2 for a in range(_A)])
    t_th = _rows(T, [85 * a + 3 for a in range(_A)])
    t_to = _rows(T, [85 * a + 4 for a in range(_A)])

    psx = jax.nn.sigmoid(p_tx)
    psy = jax.nn.sigmoid(p_ty)
    pw = jnp.exp(p_tw) * aw
    ph = jnp.exp(p_th) * ah
    gw = jnp.exp(t_tw) * aw
    gh = jnp.exp(t_th) * ah

    px1 = psx - pw * 0.5
    px2 = psx + pw * 0.5
    py1 = psy - ph * 0.5
    py2 = psy + ph * 0.5
    parea = (px2 - px1) * (py2 - py1)

    gx1 = t_tx - gw * 0.5
    gx2 = t_tx + gw * 0.5
    gy1 = t_ty - gh * 0.5
    gy2 = t_ty + gh * 0.5
    garea = (gx2 - gx1) * (gy2 - gy1)

    takenf = jnp.zeros(p_tx.shape, dtype=jnp.float32)
    updfs = []
    for g in range(_A):
        ix1 = jnp.maximum(px1, gx1[g:g + 1, :])
        iy1 = jnp.maximum(py1, gy1[g:g + 1, :])
        ix2 = jnp.minimum(px2, gx2[g:g + 1, :])
        iy2 = jnp.minimum(py2, gy2[g:g + 1, :])
        iw = jnp.maximum(ix2 - ix1, 0.0)
        ih = jnp.maximum(iy2 - iy1, 0.0)
        inter = iw * ih
        union = parea + garea[g:g + 1, :] - inter + 1e-09
        iou = inter / union
        iou = jnp.where(takenf > 0.5, -1.0, iou)
        m = jnp.max(iou, axis=0, keepdims=True)
        e = (iou == m).astype(jnp.float32)
        seen = e[0:1, :]
        rows = [seen]
        for a in range(1, _A):
            rows.append(e[a:a + 1, :] * (1.0 - seen))
            seen = jnp.maximum(seen, e[a:a + 1, :])
        oh = jnp.concatenate(rows, axis=0)
        isobj = (t_to[g:g + 1, :] > 0.5).astype(jnp.float32)
        upd = oh * isobj
        takenf = takenf + upd
        updfs.append(upd)

    objf = takenf

    def gather_tgt(tch):
        s = updfs[0] * tch[0:1, :]
        for g in range(1, _A):
            s = s + updfs[g] * tch[g:g + 1, :]
        return s

    alx = gather_tgt(t_tx)
    aly = gather_tgt(t_ty)
    alw = gather_tgt(t_tw)
    alh = gather_tgt(t_th)

    sxy = jnp.sum(objf * ((psx - alx) ** 2 + (psy - aly) ** 2))
    swh = jnp.sum(objf * ((p_tw - alw) ** 2 + (p_th - alh) ** 2))
    sobj = jnp.sum(objf * _softplus(-p_to))
    snoobj = jnp.sum((1.0 - objf) * _softplus(p_to))

    iota = jax.lax.broadcasted_iota(
        jnp.int32, (_NCLS, p_tx.shape[1]), 0).astype(jnp.float32)
    gidx = []
    for g in range(_A):
        tc = T[85 * g + 5:85 * g + 85, :]
        am = jnp.max(tc, axis=0, keepdims=True)
        gidx.append(jnp.min(jnp.where(tc == am, iota, float(_NCLS)),
                            axis=0, keepdims=True))
    cls_acc = None
    for a in range(_A):
        pc = P[85 * a + 5:85 * a + 85, :]
        m = jnp.max(pc, axis=0, keepdims=True)
        se = jnp.sum(jnp.exp(pc - m), axis=0, keepdims=True)
        lse = m + jnp.log(se)
        idx = updfs[0][a:a + 1, :] * gidx[0]
        for g in range(1, _A):
            idx = idx + updfs[g][a:a + 1, :] * gidx[g]
        psel = jnp.sum(jnp.where(iota == idx, pc, 0.0), axis=0, keepdims=True)
        term = objf[a:a + 1, :] * (lse - psel)
        cls_acc = term if cls_acc is None else cls_acc + term
    scls = jnp.sum(cls_acc)

    @pl.when(pl.program_id(0) == 0)
    def _init():
        for i in range(5):
            o_ref[i] = 0.0

    o_ref[0] += sxy
    o_ref[1] += swh
    o_ref[2] += sobj
    o_ref[3] += snoobj
    o_ref[4] += scls


def _make_call(C, CB, interpret=False):
    grid = C // CB
    return pl.pallas_call(
        _loss_body,
        grid=(grid,),
        in_specs=[
            pl.BlockSpec((CB, _CH * _A), lambda i: (i, 0)),
            pl.BlockSpec((CB, _CH * _A), lambda i: (i, 0)),
        ],
        out_specs=pl.BlockSpec(memory_space=pltpu.SMEM),
        out_shape=jax.ShapeDtypeStruct((5,), jnp.float32),
        interpret=interpret,
    )


def kernel(pred, target):
    N, S, _, A, ch = pred.shape
    C = N * S * S
    pv = pred.reshape(C, A * ch)
    tv = target.reshape(C, A * ch)
    CB = 1664 if C % 1664 == 0 else C
    sums = _make_call(C, CB)(pv, tv)
    n = jnp.float32(N)
    lxy = 5.0 * sums[0] / n
    lwh = 5.0 * sums[1] / n
    lobj = 1.0 * sums[2] / n
    lnoobj = 0.5 * sums[3] / n
    lcls = 1.0 * sums[4] / n
    total = lxy + lwh + lobj + lnoobj + lcls
    return (total, lxy, lwh, lobj, lnoobj, lcls)


# speedup vs baseline: 6.9771x; 1.6965x over previous
import jax
import jax.numpy as jnp
import numpy as np
from jax.experimental import pallas as pl
from jax.experimental.pallas import tpu as pltpu

_ANCHORS = np.array(
    [[0.57273, 0.677385], [1.87446, 2.06253], [3.33843, 5.47434],
     [7.88282, 3.52778], [9.77052, 9.16828]], dtype=np.float32)

_A = 5
_CH = 85
_NCLS = _CH - 5


def _softplus(x):
    return jnp.maximum(x, 0.0) + jnp.log1p(jnp.exp(-jnp.abs(x)))


def _rows(mat, rows_idx):
    return jnp.concatenate([mat[r:r + 1, :] for r in rows_idx], axis=0)


def _loss_body(p_ref, t_ref, o_ref):
    shp = p_ref.shape
    cells = shp[0] * shp[1] * shp[2]
    Pn = p_ref[...].reshape(cells, _A * _CH)
    Tn = t_ref[...].reshape(cells, _A * _CH)
    P = jnp.transpose(Pn, (1, 0))
    T = jnp.transpose(Tn, (1, 0))

    aw = jnp.concatenate(
        [jnp.full((1, 1), float(v), jnp.float32) for v in _ANCHORS[:, 0]], axis=0)
    ah = jnp.concatenate(
        [jnp.full((1, 1), float(v), jnp.float32) for v in _ANCHORS[:, 1]], axis=0)

    p_tx = _rows(P, [85 * a + 0 for a in range(_A)])
    p_ty = _rows(P, [85 * a + 1 for a in range(_A)])
    p_tw = _rows(P, [85 * a + 2 for a in range(_A)])
    p_th = _rows(P, [85 * a + 3 for a in range(_A)])
    p_to = _rows(P, [85 * a + 4 for a in range(_A)])
    t_tx = _rows(T, [85 * a + 0 for a in range(_A)])
    t_ty = _rows(T, [85 * a + 1 for a in range(_A)])
    t_tw = _rows(T, [85 * a + 2 for a in range(_A)])
    t_th = _rows(T, [85 * a + 3 for a in range(_A)])
    t_to = _rows(T, [85 * a + 4 for a in range(_A)])

    psx = jax.nn.sigmoid(p_tx)
    psy = jax.nn.sigmoid(p_ty)
    pw = jnp.exp(p_tw) * aw
    ph = jnp.exp(p_th) * ah
    gw = jnp.exp(t_tw) * aw
    gh = jnp.exp(t_th) * ah

    px1 = psx - pw * 0.5
    px2 = psx + pw * 0.5
    py1 = psy - ph * 0.5
    py2 = psy + ph * 0.5
    parea = (px2 - px1) * (py2 - py1)

    gx1 = t_tx - gw * 0.5
    gx2 = t_tx + gw * 0.5
    gy1 = t_ty - gh * 0.5
    gy2 = t_ty + gh * 0.5
    garea = (gx2 - gx1) * (gy2 - gy1)

    takenf = jnp.zeros(p_tx.shape, dtype=jnp.float32)
    updfs = []
    for g in range(_A):
        ix1 = jnp.maximum(px1, gx1[g:g + 1, :])
        iy1 = jnp.maximum(py1, gy1[g:g + 1, :])
        ix2 = jnp.minimum(px2, gx2[g:g + 1, :])
        iy2 = jnp.minimum(py2, gy2[g:g + 1, :])
        iw = jnp.maximum(ix2 - ix1, 0.0)
        ih = jnp.maximum(iy2 - iy1, 0.0)
        inter = iw * ih
        union = parea + garea[g:g + 1, :] - inter + 1e-09
        iou = inter / union
        iou = jnp.where(takenf > 0.5, -1.0, iou)
        m = jnp.max(iou, axis=0, keepdims=True)
        e = (iou == m).astype(jnp.float32)
        seen = e[0:1, :]
        rows = [seen]
        for a in range(1, _A):
            rows.append(e[a:a + 1, :] * (1.0 - seen))
            seen = jnp.maximum(seen, e[a:a + 1, :])
        oh = jnp.concatenate(rows, axis=0)
        isobj = (t_to[g:g + 1, :] > 0.5).astype(jnp.float32)
        upd = oh * isobj
        takenf = takenf + upd
        updfs.append(upd)

    objf = takenf

    def gather_tgt(tch):
        s = updfs[0] * tch[0:1, :]
        for g in range(1, _A):
            s = s + updfs[g] * tch[g:g + 1, :]
        return s

    alx = gather_tgt(t_tx)
    aly = gather_tgt(t_ty)
    alw = gather_tgt(t_tw)
    alh = gather_tgt(t_th)

    sxy = jnp.sum(objf * ((psx - alx) ** 2 + (psy - aly) ** 2))
    swh = jnp.sum(objf * ((p_tw - alw) ** 2 + (p_th - alh) ** 2))
    sobj = jnp.sum(objf * _softplus(-p_to))
    snoobj = jnp.sum((1.0 - objf) * _softplus(p_to))

    iota = jax.lax.broadcasted_iota(
        jnp.int32, (_NCLS, p_tx.shape[1]), 0).astype(jnp.float32)
    gidx = []
    for g in range(_A):
        tc = T[85 * g + 5:85 * g + 85, :]
        am = jnp.max(tc, axis=0, keepdims=True)
        gidx.append(jnp.min(jnp.where(tc == am, iota, float(_NCLS)),
                            axis=0, keepdims=True))
    cls_acc = None
    for a in range(_A):
        pc = P[85 * a + 5:85 * a + 85, :]
        m = jnp.max(pc, axis=0, keepdims=True)
        se = jnp.sum(jnp.exp(pc - m), axis=0, keepdims=True)
        lse = m + jnp.log(se)
        idx = updfs[0][a:a + 1, :] * gidx[0]
        for g in range(1, _A):
            idx = idx + updfs[g][a:a + 1, :] * gidx[g]
        psel = jnp.sum(jnp.where(iota == idx, pc, 0.0), axis=0, keepdims=True)
        term = objf[a:a + 1, :] * (lse - psel)
        cls_acc = term if cls_acc is None else cls_acc + term
    scls = jnp.sum(cls_acc)

    @pl.when(pl.program_id(0) == 0)
    def _init():
        for i in range(5):
            o_ref[i] = 0.0

    o_ref[0] += sxy
    o_ref[1] += swh
    o_ref[2] += sobj
    o_ref[3] += snoobj
    o_ref[4] += scls


def _make_call(N, S, NB, interpret=False):
    blk = (NB, S, S, _A, _CH)
    return pl.pallas_call(
        _loss_body,
        grid=(N // NB,),
        in_specs=[
            pl.BlockSpec(blk, lambda i: (i, 0, 0, 0, 0)),
            pl.BlockSpec(blk, lambda i: (i, 0, 0, 0, 0)),
        ],
        out_specs=pl.BlockSpec(memory_space=pltpu.SMEM),
        out_shape=jax.ShapeDtypeStruct((5,), jnp.float32),
        interpret=interpret,
    )


def kernel(pred, target):
    N, S, _, A, ch = pred.shape
    sums = _make_call(N, S, 1)(pred, target)
    n = jnp.float32(N)
    lxy = 5.0 * sums[0] / n
    lwh = 5.0 * sums[1] / n
    lobj = 1.0 * sums[2] / n
    lnoobj = 0.5 * sums[3] / n
    lcls = 1.0 * sums[4] / n
    total = lxy + lwh + lobj + lnoobj + lcls
    return (total, lxy, lwh, lobj, lnoobj, lcls)


# speedup vs baseline: 20.7751x; 2.9776x over previous
import jax
import jax.numpy as jnp
import numpy as np
from jax.experimental import pallas as pl
from jax.experimental.pallas import tpu as pltpu

_ANCHORS = np.array(
    [[0.57273, 0.677385], [1.87446, 2.06253], [3.33843, 5.47434],
     [7.88282, 3.52778], [9.77052, 9.16828]], dtype=np.float32)

_A = 5
_CH = 85
_NCLS = _CH - 5


def _softplus(x):
    return jnp.maximum(x, 0.0) + jnp.log1p(jnp.exp(-jnp.abs(x)))


def _loss_body(*refs):
    o_ref = refs[-1]
    Ps = []
    Ts = []
    for a in range(_A):
        blk = refs[a][...]
        cb = blk.shape[0] * blk.shape[1] * blk.shape[2]
        Ps.append(jnp.transpose(blk.reshape(cb, _CH), (1, 0)))
    for a in range(_A):
        blk = refs[_A + a][...]
        cb = blk.shape[0] * blk.shape[1] * blk.shape[2]
        Ts.append(jnp.transpose(blk.reshape(cb, _CH), (1, 0)))

    def chrow(mats, c):
        return jnp.concatenate([m[c:c + 1, :] for m in mats], axis=0)

    aw = jnp.concatenate(
        [jnp.full((1, 1), float(v), jnp.float32) for v in _ANCHORS[:, 0]], axis=0)
    ah = jnp.concatenate(
        [jnp.full((1, 1), float(v), jnp.float32) for v in _ANCHORS[:, 1]], axis=0)

    p_tx = chrow(Ps, 0)
    p_ty = chrow(Ps, 1)
    p_tw = chrow(Ps, 2)
    p_th = chrow(Ps, 3)
    p_to = chrow(Ps, 4)
    t_tx = chrow(Ts, 0)
    t_ty = chrow(Ts, 1)
    t_tw = chrow(Ts, 2)
    t_th = chrow(Ts, 3)
    t_to = chrow(Ts, 4)

    psx = jax.nn.sigmoid(p_tx)
    psy = jax.nn.sigmoid(p_ty)
    pw = jnp.exp(p_tw) * aw
    ph = jnp.exp(p_th) * ah
    gw = jnp.exp(t_tw) * aw
    gh = jnp.exp(t_th) * ah

    px1 = psx - pw * 0.5
    px2 = psx + pw * 0.5
    py1 = psy - ph * 0.5
    py2 = psy + ph * 0.5
    parea = (px2 - px1) * (py2 - py1)

    gx1 = t_tx - gw * 0.5
    gx2 = t_tx + gw * 0.5
    gy1 = t_ty - gh * 0.5
    gy2 = t_ty + gh * 0.5
    garea = (gx2 - gx1) * (gy2 - gy1)

    takenf = jnp.zeros(p_tx.shape, dtype=jnp.float32)
    updfs = []
    for g in range(_A):
        ix1 = jnp.maximum(px1, gx1[g:g + 1, :])
        iy1 = jnp.maximum(py1, gy1[g:g + 1, :])
        ix2 = jnp.minimum(px2, gx2[g:g + 1, :])
        iy2 = jnp.minimum(py2, gy2[g:g + 1, :])
        iw = jnp.maximum(ix2 - ix1, 0.0)
        ih = jnp.maximum(iy2 - iy1, 0.0)
        inter = iw * ih
        union = parea + garea[g:g + 1, :] - inter + 1e-09
        iou = inter / union
        iou = jnp.where(takenf > 0.5, -1.0, iou)
        m = jnp.max(iou, axis=0, keepdims=True)
        e = (iou == m).astype(jnp.float32)
        seen = e[0:1, :]
        rows = [seen]
        for a in range(1, _A):
            rows.append(e[a:a + 1, :] * (1.0 - seen))
            seen = jnp.maximum(seen, e[a:a + 1, :])
        oh = jnp.concatenate(rows, axis=0)
        isobj = (t_to[g:g + 1, :] > 0.5).astype(jnp.float32)
        upd = oh * isobj
        takenf = takenf + upd
        updfs.append(upd)

    objf = takenf

    def gather_tgt(tch):
        s = updfs[0] * tch[0:1, :]
        for g in range(1, _A):
            s = s + updfs[g] * tch[g:g + 1, :]
        return s

    alx = gather_tgt(t_tx)
    aly = gather_tgt(t_ty)
    alw = gather_tgt(t_tw)
    alh = gather_tgt(t_th)

    sxy = jnp.sum(objf * ((psx - alx) ** 2 + (psy - aly) ** 2))
    swh = jnp.sum(objf * ((p_tw - alw) ** 2 + (p_th - alh) ** 2))
    sobj = jnp.sum(objf * _softplus(-p_to))
    snoobj = jnp.sum((1.0 - objf) * _softplus(p_to))

    iota = jax.lax.broadcasted_iota(
        jnp.int32, (_NCLS, p_tx.shape[1]), 0).astype(jnp.float32)
    gidx = []
    for g in range(_A):
        tc = Ts[g][5:_CH, :]
        am = jnp.max(tc, axis=0, keepdims=True)
        gidx.append(jnp.min(jnp.where(tc == am, iota, float(_NCLS)),
                            axis=0, keepdims=True))
    cls_acc = None
    for a in range(_A):
        pc = Ps[a][5:_CH, :]
        m = jnp.max(pc, axis=0, keepdims=True)
        se = jnp.sum(jnp.exp(pc - m), axis=0, keepdims=True)
        lse = m + jnp.log(se)
        idx = updfs[0][a:a + 1, :] * gidx[0]
        for g in range(1, _A):
            idx = idx + updfs[g][a:a + 1, :] * gidx[g]
        psel = jnp.sum(jnp.where(iota == idx, pc, 0.0), axis=0, keepdims=True)
        term = objf[a:a + 1, :] * (lse - psel)
        cls_acc = term if cls_acc is None else cls_acc + term
    scls = jnp.sum(cls_acc)

    @pl.when(pl.program_id(0) == 0)
    def _init():
        for i in range(5):
            o_ref[i] = 0.0

    o_ref[0] += sxy
    o_ref[1] += swh
    o_ref[2] += sobj
    o_ref[3] += snoobj
    o_ref[4] += scls


def _make_call(N, SS, M, interpret=False):
    specs = []
    for a in range(_A):
        specs.append(pl.BlockSpec(
            (M, 1, N, _CH), lambda i, _a=a: (i, _a, 0, 0)))
    specs = specs + [pl.BlockSpec(
        (M, 1, N, _CH), lambda i, _a=a: (i, _a, 0, 0)) for a in range(_A)]
    return pl.pallas_call(
        _loss_body,
        grid=(SS // M,),
        in_specs=specs,
        out_specs=pl.BlockSpec(memory_space=pltpu.SMEM),
        out_shape=jax.ShapeDtypeStruct((5,), jnp.float32),
        interpret=interpret,
    )


def kernel(pred, target):
    N, S, _, A, ch = pred.shape
    SS = S * S
    pv = jnp.transpose(pred, (1, 2, 3, 0, 4)).reshape(SS, A, N, ch)
    tv = jnp.transpose(target, (1, 2, 3, 0, 4)).reshape(SS, A, N, ch)
    M = 52 if SS % 52 == 0 else SS
    args = [pv] * _A + [tv] * _A
    sums = _make_call(N, SS, M)(*args)
    n = jnp.float32(N)
    lxy = 5.0 * sums[0] / n
    lwh = 5.0 * sums[1] / n
    lobj = 1.0 * sums[2] / n
    lnoobj = 0.5 * sums[3] / n
    lcls = 1.0 * sums[4] / n
    total = lxy + lwh + lobj + lnoobj + lcls
    return (total, lxy, lwh, lobj, lnoobj, lcls)
